# baseline (speedup 1.0000x reference)
import jax
import jax.numpy as jnp
from jax.experimental import pallas as pl

MP_ = 65536


def kernel(inputs, mat_rows, mat_cols, mat_vals):
    Nb, Mm, Fin = inputs.shape
    x = jnp.transpose(inputs, (1, 2, 0)).reshape(Mm, -1)
    gathered = mat_vals[:, None] * jnp.take(x, mat_cols, axis=0)
    y = jax.ops.segment_sum(gathered, mat_rows, num_segments=MP_)
    y = y.reshape(MP_, Fin, -1)
    return jnp.transpose(y, (2, 0, 1))


# speedup vs baseline: 1.9049x; 1.9049x over previous
import jax
import jax.numpy as jnp
from jax import lax
from jax.experimental import pallas as pl
from jax.experimental.pallas import tpu as pltpu
from jax.experimental.pallas import tpu_sc as plsc

N = 4
M = 16384
MP = 65536
NNZ = 262144
F = 64
PB = N // 2
F2 = 2 * F
NC = 2
NS = 16
L = 16
RB = 8192
NBLK = MP // RB
GPT = 16
CHUNK = GPT * L
PAIRS = (PB * NBLK) // NC
ZR = RB // NS


def _body(xp, cols, vals, rows, bp, zin, out,
          shared, bp_v, cols_v, vals_v, rows_v, xg_v, contrib_v,
          idx_sem, xg_sem):
    c = lax.axis_index("c")
    s = lax.axis_index("s")
    iota = lax.iota(jnp.int32, L)
    zv = jnp.zeros((L,), jnp.int32)
    pltpu.sync_copy(bp, bp_v)

    for p in range(PAIRS):
        u = p * NC + c
        pb = u // NBLK
        b = u % NBLK
        bpall = bp_v[pl.ds(0, L)]
        start = jnp.sum(jnp.where(iota == b, bpall, 0))
        end = jnp.sum(jnp.where(iota == b + 1, bpall, 0))
        g0 = start // L
        g1 = (end + (L - 1)) // L
        mt = (g1 - g0 + (NS - 1)) // NS
        gb = g0 + s * mt
        ge = jnp.minimum(gb + mt, g1)
        rowoff = b * RB
        start_v = zv + start
        end_v = zv + end

        pltpu.sync_copy(zin, shared.at[pl.ds(s * ZR, ZR)])
        plsc.subcore_barrier()

        def trip(t):
            psc = jnp.minimum((gb + t * GPT) * L, NNZ - CHUNK)
            d1 = pltpu.async_copy(cols.at[pl.ds(psc, CHUNK)], cols_v, idx_sem)
            d2 = pltpu.async_copy(vals.at[pl.ds(psc, CHUNK)], vals_v, idx_sem)
            d3 = pltpu.async_copy(rows.at[pl.ds(psc, CHUNK)], rows_v, idx_sem)
            d1.wait()
            d2.wait()
            d3.wait()
            xds = []
            for j in range(GPT):
                colj = plsc.load_gather(cols_v, [j * L + iota])
                xds.append(pltpu.async_copy(
                    xp.at[colj + pb * M], xg_v.at[pl.ds(j * L, L)], xg_sem))
            for d in xds:
                d.wait()

            lo_v = jnp.maximum(start_v, zv + (gb + t * GPT) * L)
            hi_v = jnp.minimum(end_v, zv + ge * L)

            def grp(j, carry):
                pos = zv + psc + j * L + iota
                msk = (pos >= lo_v) & (pos < hi_v)
                rowj = plsc.load_gather(rows_v, [j * L + iota])
                lidx = jnp.where(msk, rowj - rowoff, RB + s)
                for k in range(L):
                    kv = zv + (j * L + k)
                    vb = plsc.load_gather(vals_v, [kv])
                    for q in range(F2 // L):
                        xrow = plsc.load_gather(xg_v, [kv, q * L + iota])
                        contrib_v[k, pl.ds(q * L, L)] = vb * xrow
                pltpu.sync_copy(contrib_v, shared.at[lidx], add=True)
                return carry

            lax.fori_loop(0, GPT, grp, 0)
            return t + 1

        lax.while_loop(lambda t: gb + t * GPT < ge, trip, jnp.int32(0))
        plsc.subcore_barrier()
        pltpu.sync_copy(shared.at[pl.ds(s * ZR, ZR)],
                        out.at[pl.ds(pb * MP + rowoff + s * ZR, ZR)])
        plsc.subcore_barrier()


_spmm = pl.kernel(
    _body,
    out_type=jax.ShapeDtypeStruct((PB * MP, F2), jnp.float32),
    mesh=plsc.VectorSubcoreMesh(core_axis_name="c", subcore_axis_name="s"),
    compiler_params=pltpu.CompilerParams(needs_layout_passes=False),
    scratch_types=[
        pltpu.VMEM_SHARED((RB + NS, F2), jnp.float32),
        pltpu.VMEM((32,), jnp.int32),
        pltpu.VMEM((CHUNK,), jnp.int32),
        pltpu.VMEM((CHUNK,), jnp.float32),
        pltpu.VMEM((CHUNK,), jnp.int32),
        pltpu.VMEM((CHUNK, F2), jnp.float32),
        pltpu.VMEM((L, F2), jnp.float32),
        pltpu.SemaphoreType.DMA,
        pltpu.SemaphoreType.DMA,
    ],
)


@jax.jit
def kernel(inputs, mat_rows, mat_cols, mat_vals):
    xp = inputs.reshape(PB, 2, M, F).transpose(0, 2, 1, 3).reshape(PB * M, F2)
    bounds = jnp.arange(0, MP + 1, RB, dtype=jnp.int32)
    bp1 = jnp.searchsorted(mat_rows, bounds, side="left").astype(jnp.int32)
    bp = jnp.concatenate([bp1, jnp.zeros((32 - NBLK - 1,), jnp.int32)])
    zin = jnp.zeros((ZR, F2), jnp.float32)
    y = _spmm(xp, mat_cols, mat_vals, mat_rows, bp, zin)
    return y.reshape(PB, MP, 2, F).transpose(0, 2, 1, 3).reshape(N, MP, F)


# speedup vs baseline: 2.1289x; 1.1176x over previous
import jax
import jax.numpy as jnp
from jax import lax
from jax.experimental import pallas as pl
from jax.experimental.pallas import tpu as pltpu
from jax.experimental.pallas import tpu_sc as plsc

N = 4
M = 16384
MP = 65536
NNZ = 262144
F = 64
PB = N // 2
F2 = 2 * F
NC = 2
NS = 16
L = 16
RB = 4096
NBLK = MP // RB
GPT = 16
HG = GPT // 2
CHUNK = GPT * L
HROW = HG * L
PAIRS = (PB * NBLK) // NC
ZR = RB // NS
ZB = 128


def _body(xp, cols, vals, rows, bp, zin, out,
          shared, bp_v, cols_v, vals_v, rows_v, xg_v, lidx_v, zbuf,
          idx_sem, xg_sem, sc_sem):
    c = lax.axis_index("c")
    s = lax.axis_index("s")
    iota = lax.iota(jnp.int32, L)
    zv = jnp.zeros((L,), jnp.int32)
    pltpu.sync_copy(bp, bp_v)
    pltpu.sync_copy(zin, zbuf)

    def fire_idx(gb, t, buf):
        psc = jnp.minimum((gb + t * GPT) * L, NNZ - CHUNK)
        pltpu.async_copy(cols.at[pl.ds(psc, CHUNK)],
                         cols_v.at[pl.ds(buf * CHUNK, CHUNK)], idx_sem)
        pltpu.async_copy(vals.at[pl.ds(psc, CHUNK)],
                         vals_v.at[pl.ds(buf * CHUNK, CHUNK)], idx_sem)
        pltpu.async_copy(rows.at[pl.ds(psc, CHUNK)],
                         rows_v.at[pl.ds(buf * CHUNK, CHUNK)], idx_sem)

    def drain_idx():
        pltpu.make_async_copy(cols.at[pl.ds(0, CHUNK)],
                              cols_v.at[pl.ds(0, CHUNK)], idx_sem).wait()
        pltpu.make_async_copy(vals.at[pl.ds(0, CHUNK)],
                              vals_v.at[pl.ds(0, CHUNK)], idx_sem).wait()
        pltpu.make_async_copy(rows.at[pl.ds(0, CHUNK)],
                              rows_v.at[pl.ds(0, CHUNK)], idx_sem).wait()

    def fire_gathers(pbase, buf):
        for j in range(GPT):
            colj = plsc.load_gather(
                cols_v, [buf * CHUNK + j * L + iota])
            pltpu.async_copy(
                xp.at[colj + pbase],
                xg_v.at[pl.ds(buf * CHUNK + j * L, L)], xg_sem)

    def drain_gathers():
        pltpu.make_async_copy(xp.at[pl.ds(0, CHUNK)],
                              xg_v.at[pl.ds(0, CHUNK)], xg_sem).wait()

    def drain_scatters():
        for h in range(2):
            pltpu.make_async_copy(
                xg_v.at[pl.ds(h * HROW, HROW)],
                shared.at[pl.ds(h * HROW, HROW)], sc_sem).wait()

    def pair_body(p, pcarry):
        u = p * NC + c
        pb = u // NBLK
        b = u % NBLK
        pbase = pb * M
        start = plsc.load_gather(bp_v, [zv + b])[0]
        end = plsc.load_gather(bp_v, [zv + (b + 1)])[0]
        g0 = start // L
        g1 = (end + (L - 1)) // L
        mt = (g1 - g0 + (NS - 1)) // NS
        gb = g0 + s * mt
        ge = jnp.minimum(gb + mt, g1)
        rowoff = b * RB
        start_v = zv + start
        end_v = zv + end
        hi_v = jnp.minimum(end_v, zv + ge * L)

        for z in range(ZR // ZB):
            pltpu.sync_copy(zbuf, shared.at[pl.ds(s * ZR + z * ZB, ZB)])
        plsc.subcore_barrier()

        pltpu.sync_copy(cols.at[pl.ds(jnp.minimum(gb * L, NNZ - CHUNK),
                                      CHUNK)], cols_v.at[pl.ds(0, CHUNK)])
        pltpu.sync_copy(vals.at[pl.ds(jnp.minimum(gb * L, NNZ - CHUNK),
                                      CHUNK)], vals_v.at[pl.ds(0, CHUNK)])
        pltpu.sync_copy(rows.at[pl.ds(jnp.minimum(gb * L, NNZ - CHUNK),
                                      CHUNK)], rows_v.at[pl.ds(0, CHUNK)])
        fire_gathers(pbase, 0)
        fire_idx(gb, 1, 1)

        def trip(t):
            buf = t % 2
            nbuf = 1 - buf
            psc = jnp.minimum((gb + t * GPT) * L, NNZ - CHUNK)
            lo_v = jnp.maximum(start_v, zv + (gb + t * GPT) * L)

            drain_gathers()
            drain_idx()

            @pl.when(t > 0)
            def _():
                drain_scatters()

            fire_gathers(pbase, nbuf)

            def grp(jj, carry):
                for h in range(2):
                    j = jj + h * HG
                    co = buf * CHUNK + j * L
                    pos = zv + psc + j * L + iota
                    msk = (pos >= lo_v) & (pos < hi_v)
                    rowj = plsc.load_gather(rows_v, [co + iota])
                    lidx = jnp.where(msk, rowj - rowoff, RB + s)
                    plsc.store_scatter(
                        lidx_v, [zv + (buf * 2 + h), jj * L + iota], lidx)
                    for k in range(L):
                        kv = zv + (co + k)
                        vb = plsc.load_gather(vals_v, [kv])
                        for q in range(F2 // L):
                            cix = q * L + iota
                            xrow = plsc.load_gather(xg_v, [kv, cix])
                            plsc.store_scatter(xg_v, [kv, cix], vb * xrow)
                return carry

            lax.fori_loop(0, HG, grp, 0)
            for h in range(2):
                pltpu.async_copy(
                    xg_v.at[pl.ds(buf * CHUNK + h * HROW, HROW)],
                    shared.at[lidx_v.at[buf * 2 + h]],
                    sc_sem, add=True)
            fire_idx(gb, t + 2, buf)
            return t + 1

        tT = lax.while_loop(lambda t: gb + t * GPT < ge, trip, jnp.int32(0))
        drain_gathers()
        drain_idx()

        @pl.when(tT > 0)
        def _():
            drain_scatters()
        plsc.subcore_barrier()
        pltpu.sync_copy(shared.at[pl.ds(s * ZR, ZR)],
                        out.at[pl.ds(pb * MP + rowoff + s * ZR, ZR)])
        plsc.subcore_barrier()
        return pcarry

    lax.fori_loop(0, PAIRS, pair_body, 0)


_spmm = pl.kernel(
    _body,
    out_type=jax.ShapeDtypeStruct((PB * MP, F2), jnp.float32),
    mesh=plsc.VectorSubcoreMesh(core_axis_name="c", subcore_axis_name="s"),
    compiler_params=pltpu.CompilerParams(needs_layout_passes=False),
    scratch_types=[
        pltpu.VMEM_SHARED((RB + NS, F2), jnp.float32),
        pltpu.VMEM((32,), jnp.int32),
        pltpu.VMEM((2 * CHUNK,), jnp.int32),
        pltpu.VMEM((2 * CHUNK,), jnp.float32),
        pltpu.VMEM((2 * CHUNK,), jnp.int32),
        pltpu.VMEM((2 * CHUNK, F2), jnp.float32),
        pltpu.VMEM((4, HROW), jnp.int32),
        pltpu.VMEM((ZB, F2), jnp.float32),
        pltpu.SemaphoreType.DMA,
        pltpu.SemaphoreType.DMA,
        pltpu.SemaphoreType.DMA,
    ],
)


@jax.jit
def kernel(inputs, mat_rows, mat_cols, mat_vals):
    xp = inputs.reshape(PB, 2, M, F).transpose(0, 2, 1, 3).reshape(PB * M, F2)
    bounds = jnp.arange(0, MP + 1, RB, dtype=jnp.int32)
    bp1 = jnp.searchsorted(mat_rows, bounds, side="left").astype(jnp.int32)
    bp = jnp.concatenate([bp1, jnp.zeros((32 - NBLK - 1,), jnp.int32)])
    zin = jnp.zeros((ZB, F2), jnp.float32)
    y = _spmm(xp, mat_cols, mat_vals, mat_rows, bp, zin)
    return y.reshape(PB, MP, 2, F).transpose(0, 2, 1, 3).reshape(N, MP, F)


# speedup vs baseline: 2.9142x; 1.3689x over previous
import jax
import jax.numpy as jnp
from jax import lax
from jax.experimental import pallas as pl
from jax.experimental.pallas import tpu as pltpu
from jax.experimental.pallas import tpu_sc as plsc

N = 4
M = 16384
MP = 65536
NNZ = 262144
F = 64
F2 = 2 * F
NC = 2
NS = 16
L = 16
RB = 4096
NBLK = MP // RB
GPT = 8
HG = GPT // 2
CHUNK = GPT * L
HROW = HG * L
ZR = RB // NS
PKR = 32
RPT = M // NS
FQ = F // L
BPAD = 64


def _body(xin, cols, vals, rows, bp, y, xp,
          shared, bp_v, cols_v, vals_v, rows_v, xg_v, sc_v, lidx_v, ca_v,
          po_v, pa_v, pb_v, st_v,
          idx_sem, xg_sem, sc_sem):
    c = lax.axis_index("c")
    s = lax.axis_index("s")
    iota = lax.iota(jnp.int32, L)
    zv = jnp.zeros((L,), jnp.int32)
    pltpu.sync_copy(bp, bp_v)
    pbase = c * M

    def pack(w, carry):
        m0 = s * RPT + w * PKR
        pltpu.sync_copy(xin.at[2 * c, pl.ds(m0, PKR), :], pa_v)
        pltpu.sync_copy(xin.at[2 * c + 1, pl.ds(m0, PKR), :], pb_v)
        for r in range(PKR):
            for q in range(FQ):
                po_v[r, pl.ds(q * L, L)] = pa_v[r, pl.ds(q * L, L)]
                po_v[r, pl.ds(F + q * L, L)] = pb_v[r, pl.ds(q * L, L)]
        pltpu.sync_copy(po_v, xp.at[pl.ds(pbase + m0, PKR)])
        return carry

    lax.fori_loop(0, RPT // PKR, pack, 0)
    for r in range(PKR):
        for q in range(F2 // L):
            po_v[r, pl.ds(q * L, L)] = jnp.zeros((L,), jnp.float32)
    plsc.subcore_barrier()

    def fire_idx(gb, t, buf):
        psc = jnp.minimum((gb + t * GPT) * L, NNZ - CHUNK)
        pltpu.async_copy(cols.at[pl.ds(psc, CHUNK)],
                         cols_v.at[pl.ds(buf * CHUNK, CHUNK)], idx_sem)
        pltpu.async_copy(vals.at[pl.ds(psc, CHUNK)],
                         vals_v.at[pl.ds(buf * CHUNK, CHUNK)], idx_sem)
        pltpu.async_copy(rows.at[pl.ds(psc, CHUNK)],
                         rows_v.at[pl.ds(buf * CHUNK, CHUNK)], idx_sem)

    def drain_idx():
        pltpu.make_async_copy(cols.at[pl.ds(0, CHUNK)],
                              cols_v.at[pl.ds(0, CHUNK)], idx_sem).wait()
        pltpu.make_async_copy(vals.at[pl.ds(0, CHUNK)],
                              vals_v.at[pl.ds(0, CHUNK)], idx_sem).wait()
        pltpu.make_async_copy(rows.at[pl.ds(0, CHUNK)],
                              rows_v.at[pl.ds(0, CHUNK)], idx_sem).wait()

    def fire_gathers(buf):
        for u in range(CHUNK // L):
            colu = plsc.load_gather(cols_v, [buf * CHUNK + u * L + iota])
            plsc.store_scatter(ca_v, [buf * CHUNK + u * L + iota],
                               colu + pbase)
        for u4 in range(4):
            pltpu.async_copy(
                xp.at[ca_v.at[pl.ds(buf * CHUNK + u4 * (CHUNK // 4),
                                    CHUNK // 4)]],
                xg_v.at[pl.ds(buf * CHUNK + u4 * (CHUNK // 4), CHUNK // 4)],
                xg_sem)

    def drain_gathers():
        pltpu.make_async_copy(xp.at[pl.ds(0, CHUNK)],
                              xg_v.at[pl.ds(0, CHUNK)], xg_sem).wait()

    def drain_scatters():
        pltpu.make_async_copy(sc_v, shared.at[pl.ds(0, CHUNK)],
                              sc_sem).wait()

    def pair_body(b, pcarry):
        start = plsc.load_gather(bp_v, [zv + b])[0]
        end = plsc.load_gather(bp_v, [zv + (b + 1)])[0]
        g0 = start // L
        g1 = (end + (L - 1)) // L
        mt = (g1 - g0 + (NS - 1)) // NS
        gb = g0 + s * mt
        ge = jnp.minimum(gb + mt, g1)
        rowoff = b * RB
        start_v = zv + start
        end_v = zv + end
        hi_v = jnp.minimum(end_v, zv + ge * L)

        for z in range(ZR // PKR):
            pltpu.sync_copy(po_v, shared.at[pl.ds(s * ZR + z * PKR, PKR)])
        plsc.subcore_barrier()

        p0 = jnp.minimum(gb * L, NNZ - CHUNK)
        pltpu.sync_copy(cols.at[pl.ds(p0, CHUNK)], cols_v.at[pl.ds(0, CHUNK)])
        pltpu.sync_copy(vals.at[pl.ds(p0, CHUNK)], vals_v.at[pl.ds(0, CHUNK)])
        pltpu.sync_copy(rows.at[pl.ds(p0, CHUNK)], rows_v.at[pl.ds(0, CHUNK)])
        fire_gathers(0)
        fire_idx(gb, 1, 1)

        def trip(t):
            buf = t % 2
            nbuf = 1 - buf
            psc = jnp.minimum((gb + t * GPT) * L, NNZ - CHUNK)
            lo_v = jnp.maximum(start_v, zv + (gb + t * GPT) * L)

            drain_gathers()
            drain_idx()

            @pl.when(t > 0)
            def _():
                drain_scatters()

            fire_gathers(nbuf)

            @plsc.parallel_loop(0, GPT, 1, unroll=2, carry=jnp.int32(0))
            def grp(j, carry):
                co = buf * CHUNK + j * L
                pos = zv + psc + j * L + iota
                msk = (pos >= lo_v) & (pos < hi_v)
                rowj = plsc.load_gather(rows_v, [co + iota])
                lidx = jnp.where(msk, rowj - rowoff, RB + s)
                plsc.store_scatter(lidx_v, [zv + buf, j * L + iota], lidx)
                for k in range(L):
                    kv = zv + (co + k)
                    cv = zv + (j * L + k)
                    vb = plsc.load_gather(vals_v, [kv])
                    for q in range(F2 // L):
                        cix = q * L + iota
                        xrow = plsc.load_gather(xg_v, [kv, cix])
                        plsc.store_scatter(sc_v, [cv, cix], vb * xrow)
                return carry

            pltpu.async_copy(sc_v, shared.at[lidx_v.at[buf]],
                             sc_sem, add=True)
            fire_idx(gb, t + 2, buf)
            return t + 1

        tT = lax.while_loop(lambda t: gb + t * GPT < ge, trip, jnp.int32(0))
        drain_gathers()
        drain_idx()

        @pl.when(tT > 0)
        def _():
            drain_scatters()
        plsc.subcore_barrier()

        def unpk(w, carry):
            r0 = s * ZR + w * PKR
            pltpu.sync_copy(shared.at[pl.ds(r0, PKR)], st_v)
            for r in range(PKR):
                for q in range(FQ):
                    pa_v[r, pl.ds(q * L, L)] = st_v[r, pl.ds(q * L, L)]
                    pb_v[r, pl.ds(q * L, L)] = st_v[r, pl.ds(F + q * L, L)]
            ob = rowoff + r0
            pltpu.sync_copy(pa_v, y.at[2 * c, pl.ds(ob, PKR), :])
            pltpu.sync_copy(pb_v, y.at[2 * c + 1, pl.ds(ob, PKR), :])
            return carry

        lax.fori_loop(0, ZR // PKR, unpk, 0)
        plsc.subcore_barrier()
        return pcarry

    lax.fori_loop(0, NBLK, pair_body, 0)


_spmm = pl.kernel(
    _body,
    out_type=(
        jax.ShapeDtypeStruct((N, MP, F), jnp.float32),
        jax.ShapeDtypeStruct((NC * M, F2), jnp.float32),
    ),
    mesh=plsc.VectorSubcoreMesh(core_axis_name="c", subcore_axis_name="s"),
    compiler_params=pltpu.CompilerParams(needs_layout_passes=False),
    scratch_types=[
        pltpu.VMEM_SHARED((RB + NS, F2), jnp.float32),
        pltpu.VMEM((BPAD,), jnp.int32),
        pltpu.VMEM((2 * CHUNK,), jnp.int32),
        pltpu.VMEM((2 * CHUNK,), jnp.float32),
        pltpu.VMEM((2 * CHUNK,), jnp.int32),
        pltpu.VMEM((2 * CHUNK, F2), jnp.float32),
        pltpu.VMEM((CHUNK, F2), jnp.float32),
        pltpu.VMEM((2, CHUNK), jnp.int32),
        pltpu.VMEM((2 * CHUNK,), jnp.int32),
        pltpu.VMEM((PKR, F2), jnp.float32),
        pltpu.VMEM((PKR, F), jnp.float32),
        pltpu.VMEM((PKR, F), jnp.float32),
        pltpu.VMEM((PKR, F2), jnp.float32),
        pltpu.SemaphoreType.DMA,
        pltpu.SemaphoreType.DMA,
        pltpu.SemaphoreType.DMA,
    ],
)


@jax.jit
def kernel(inputs, mat_rows, mat_cols, mat_vals):
    bounds = jnp.arange(0, MP + 1, RB, dtype=jnp.int32)
    bp1 = jnp.searchsorted(mat_rows, bounds, side="left").astype(jnp.int32)
    bp = jnp.concatenate([bp1, jnp.zeros((BPAD - NBLK - 1,), jnp.int32)])
    y, _ = _spmm(inputs, mat_cols, mat_vals, mat_rows, bp)
    return y
